# Initial kernel scaffold; baseline (speedup 1.0000x reference)
import jax
import jax.numpy as jnp
from jax.experimental import pallas as pl


def kernel(x, seg, token_table, pos_table, seg_table, gamma, beta):
    raise NotImplementedError("write your pallas kernel here")


# speedup vs baseline: 5.0546x; 5.0546x over previous
import functools

import jax
import jax.numpy as jnp
from jax import lax
from jax.experimental import pallas as pl
from jax.experimental.pallas import tpu as pltpu
from jax.experimental.pallas import tpu_sc as plsc

_NW = 32
_C = 128
_BB = 8


def _make_sc_gather(n, d):
  rpw = n // _NW
  assert n % _NW == 0 and rpw % _C == 0
  nch = rpw // _C
  assert nch % 2 == 0
  npairs = nch // 2
  mesh = plsc.VectorSubcoreMesh(core_axis_name="c", subcore_axis_name="s")

  @functools.partial(
      pl.kernel,
      out_type=jax.ShapeDtypeStruct((n, d), jnp.float32),
      mesh=mesh,
      scratch_types=[
          pltpu.VMEM((2, _C), jnp.int32),
          pltpu.VMEM((2, _C, d), jnp.float32),
          pltpu.SemaphoreType.DMA,
          pltpu.SemaphoreType.DMA,
          pltpu.SemaphoreType.DMA,
          pltpu.SemaphoreType.DMA,
      ],
  )
  def sc_gather(table_hbm, idx_hbm, out_hbm, idx_v, rows_v,
                gsem0, gsem1, wsem0, wsem1):
    wid = lax.axis_index("s") * 2 + lax.axis_index("c")
    base0 = wid * rpw
    gsems = (gsem0, gsem1)
    wsems = (wsem0, wsem1)

    def start_gather(ci, buf):
      base = base0 + ci * _C
      pltpu.sync_copy(idx_hbm.at[pl.ds(base, _C)], idx_v.at[buf])
      pltpu.async_copy(table_hbm.at[idx_v.at[buf]], rows_v.at[buf],
                       gsems[buf])

    def wait_gather(buf):
      pltpu.make_async_copy(table_hbm.at[idx_v.at[buf]], rows_v.at[buf],
                            gsems[buf]).wait()

    def start_wb(ci, buf):
      pltpu.async_copy(rows_v.at[buf],
                       out_hbm.at[pl.ds(base0 + ci * _C, _C)], wsems[buf])

    def wait_wb(ci, buf):
      pltpu.make_async_copy(rows_v.at[buf],
                            out_hbm.at[pl.ds(base0 + ci * _C, _C)],
                            wsems[buf]).wait()

    start_gather(0, 0)

    def pair(g, carry):
      a = 2 * g
      wait_gather(0)

      @pl.when(g >= 1)
      def _():
        wait_wb(a - 1, 1)

      start_gather(a + 1, 1)
      start_wb(a, 0)
      wait_gather(1)
      wait_wb(a, 0)

      @pl.when(a + 2 < nch)
      def _():
        start_gather(a + 2, 0)

      start_wb(a + 1, 1)
      return carry

    lax.fori_loop(0, npairs, pair, 0)
    wait_wb(nch - 1, 1)

  return sc_gather


def _ln_body(nseg, eps, tok_ref, segi_ref, pos_ref, segtab_ref, gam_ref,
             bet_ref, out_ref):
  tok = tok_ref[...]
  pos = pos_ref[...]
  segi = segi_ref[...]
  st = segtab_ref[...]
  h = tok + pos[None, :, :]
  seg_enc = jnp.broadcast_to(st[0][None, None, :], h.shape)
  for k in range(1, nseg):
    seg_enc = jnp.where(segi == k, st[k][None, None, :], seg_enc)
  h = h + seg_enc
  mean = jnp.mean(h, axis=-1, keepdims=True)
  c = h - mean
  var = jnp.mean(c * c, axis=-1, keepdims=True)
  out_ref[...] = (c * lax.rsqrt(var + eps) * gam_ref[...][None, :, :]
                  + bet_ref[...][None, :, :])


def _make_tc_ln(b, s, d, nseg, eps):
  assert b % _BB == 0
  grid = (b // _BB,)
  return pl.pallas_call(
      functools.partial(_ln_body, nseg, eps),
      grid=grid,
      in_specs=[
          pl.BlockSpec((_BB, s, d), lambda i: (i, 0, 0)),
          pl.BlockSpec((_BB, s, 1), lambda i: (i, 0, 0)),
          pl.BlockSpec((s, d), lambda i: (0, 0)),
          pl.BlockSpec((nseg, d), lambda i: (0, 0)),
          pl.BlockSpec((1, d), lambda i: (0, 0)),
          pl.BlockSpec((1, d), lambda i: (0, 0)),
      ],
      out_specs=pl.BlockSpec((_BB, s, d), lambda i: (i, 0, 0)),
      out_shape=jax.ShapeDtypeStruct((b, s, d), jnp.float32),
      compiler_params=pltpu.CompilerParams(
          dimension_semantics=("arbitrary",)),
  )


def kernel(x, seg, token_table, pos_table, seg_table, gamma, beta):
  b, s = x.shape
  _, d = token_table.shape
  nseg = seg_table.shape[0]
  n = b * s
  xf = x.reshape(n).astype(jnp.int32)
  gathered = _make_sc_gather(n, d)(token_table.astype(jnp.float32), xf)
  segi = seg.astype(jnp.int32)[:, :, None]
  out = _make_tc_ln(b, s, d, nseg, 1e-5)(
      gathered.reshape(b, s, d), segi, pos_table.astype(jnp.float32),
      seg_table.astype(jnp.float32), gamma.astype(jnp.float32).reshape(1, d),
      beta.astype(jnp.float32).reshape(1, d))
  return out


# speedup vs baseline: 5.2924x; 1.0470x over previous
import functools

import jax
import jax.numpy as jnp
from jax import lax
from jax.experimental import pallas as pl
from jax.experimental.pallas import tpu as pltpu
from jax.experimental.pallas import tpu_sc as plsc

_NW = 32
_C = 128
_BB = 8


def _make_sc_gather(n, d):
  rpw = n // _NW
  assert n % _NW == 0 and rpw % _C == 0
  nch = rpw // _C
  assert nch % 2 == 0
  npairs = nch // 2
  mesh = plsc.VectorSubcoreMesh(core_axis_name="c", subcore_axis_name="s")

  @functools.partial(
      pl.kernel,
      out_type=jax.ShapeDtypeStruct((n, d), jnp.float32),
      mesh=mesh,
      scratch_types=[
          pltpu.VMEM((nch, _C), jnp.int32),
          pltpu.VMEM((2, _C, d), jnp.float32),
          pltpu.SemaphoreType.DMA,
          pltpu.SemaphoreType.DMA,
          pltpu.SemaphoreType.DMA,
          pltpu.SemaphoreType.DMA,
      ],
  )
  def sc_gather(table_hbm, idx_hbm, out_hbm, idx_v, rows_v,
                gsem0, gsem1, wsem0, wsem1):
    wid = lax.axis_index("s") * 2 + lax.axis_index("c")
    base0 = wid * rpw
    gsems = (gsem0, gsem1)
    wsems = (wsem0, wsem1)
    pltpu.sync_copy(idx_hbm.at[wid], idx_v)

    def start_gather(ci, buf):
      pltpu.async_copy(table_hbm.at[idx_v.at[ci]], rows_v.at[buf],
                       gsems[buf])

    def wait_gather(ci, buf):
      pltpu.make_async_copy(table_hbm.at[idx_v.at[ci]], rows_v.at[buf],
                            gsems[buf]).wait()

    def start_wb(ci, buf):
      pltpu.async_copy(rows_v.at[buf],
                       out_hbm.at[pl.ds(base0 + ci * _C, _C)], wsems[buf])

    def wait_wb(ci, buf):
      pltpu.make_async_copy(rows_v.at[buf],
                            out_hbm.at[pl.ds(base0 + ci * _C, _C)],
                            wsems[buf]).wait()

    start_gather(0, 0)

    def pair(g, carry):
      a = 2 * g
      wait_gather(a, 0)

      @pl.when(g >= 1)
      def _():
        wait_wb(a - 1, 1)

      start_gather(a + 1, 1)
      start_wb(a, 0)
      wait_gather(a + 1, 1)
      wait_wb(a, 0)

      @pl.when(a + 2 < nch)
      def _():
        start_gather(a + 2, 0)

      start_wb(a + 1, 1)
      return carry

    lax.fori_loop(0, npairs, pair, 0)
    wait_wb(nch - 1, 1)

  return sc_gather


def _ln_body(nseg, eps, s, d, tok_ref, segi_ref, pos_ref, segtab_ref,
             gam_ref, bet_ref, out_ref):
  tok = tok_ref[...].reshape(_BB, s, d)
  pos = pos_ref[...]
  segi = segi_ref[...].reshape(_BB, s, 1)
  st = segtab_ref[...]
  h = tok + pos[None, :, :]
  seg_enc = jnp.broadcast_to(st[0][None, None, :], h.shape)
  for k in range(1, nseg):
    seg_enc = jnp.where(segi == k, st[k][None, None, :], seg_enc)
  h = h + seg_enc
  mean = jnp.mean(h, axis=-1, keepdims=True)
  c = h - mean
  var = jnp.mean(c * c, axis=-1, keepdims=True)
  out_ref[...] = (c * lax.rsqrt(var + eps) * gam_ref[...][None, :, :]
                  + bet_ref[...][None, :, :])


def _make_tc_ln(b, s, d, nseg, eps):
  assert b % _BB == 0
  grid = (b // _BB,)
  return pl.pallas_call(
      functools.partial(_ln_body, nseg, eps, s, d),
      grid=grid,
      in_specs=[
          pl.BlockSpec((_BB * s, d), lambda i: (i, 0)),
          pl.BlockSpec((_BB * s, 1), lambda i: (i, 0)),
          pl.BlockSpec((s, d), lambda i: (0, 0)),
          pl.BlockSpec((nseg, d), lambda i: (0, 0)),
          pl.BlockSpec((1, d), lambda i: (0, 0)),
          pl.BlockSpec((1, d), lambda i: (0, 0)),
      ],
      out_specs=pl.BlockSpec((_BB, s, d), lambda i: (i, 0, 0)),
      out_shape=jax.ShapeDtypeStruct((b, s, d), jnp.float32),
      compiler_params=pltpu.CompilerParams(
          dimension_semantics=("arbitrary",)),
  )


def kernel(x, seg, token_table, pos_table, seg_table, gamma, beta):
  b, s = x.shape
  _, d = token_table.shape
  nseg = seg_table.shape[0]
  n = b * s
  xf = x.reshape(_NW, n // (_NW * _C), _C).astype(jnp.int32)
  gathered = _make_sc_gather(n, d)(token_table.astype(jnp.float32), xf)
  segi = seg.astype(jnp.int32).reshape(n, 1)
  out = _make_tc_ln(b, s, d, nseg, 1e-5)(
      gathered, segi, pos_table.astype(jnp.float32),
      seg_table.astype(jnp.float32), gamma.astype(jnp.float32).reshape(1, d),
      beta.astype(jnp.float32).reshape(1, d))
  return out


# speedup vs baseline: 6.0671x; 1.1464x over previous
import functools

import jax
import jax.numpy as jnp
from jax import lax
from jax.experimental import pallas as pl
from jax.experimental.pallas import tpu as pltpu
from jax.experimental.pallas import tpu_sc as plsc

_NW = 32
_C = 128
_BB = 8


def _make_sc_gather(n, d):
  rpw = n // _NW
  assert n % _NW == 0 and rpw % _C == 0
  nch = rpw // _C
  assert nch % 2 == 0
  npairs = nch // 2
  mesh = plsc.VectorSubcoreMesh(core_axis_name="c", subcore_axis_name="s")

  @functools.partial(
      pl.kernel,
      out_type=jax.ShapeDtypeStruct((n, d), jnp.float32),
      mesh=mesh,
      scratch_types=[
          pltpu.VMEM((nch, _C), jnp.int32),
          pltpu.VMEM((2, _C, d), jnp.float32),
          pltpu.SemaphoreType.DMA,
          pltpu.SemaphoreType.DMA,
          pltpu.SemaphoreType.DMA,
          pltpu.SemaphoreType.DMA,
      ],
  )
  def sc_gather(table_hbm, idx_hbm, out_hbm, idx_v, rows_v,
                gsem0, gsem1, wsem0, wsem1):
    wid = lax.axis_index("s") * 2 + lax.axis_index("c")
    base0 = wid * rpw
    gsems = (gsem0, gsem1)
    wsems = (wsem0, wsem1)
    pltpu.sync_copy(idx_hbm.at[wid], idx_v)

    def start_gather(ci, buf):
      pltpu.async_copy(table_hbm.at[idx_v.at[ci]], rows_v.at[buf],
                       gsems[buf])

    def wait_gather(ci, buf):
      pltpu.make_async_copy(table_hbm.at[idx_v.at[ci]], rows_v.at[buf],
                            gsems[buf]).wait()

    def start_wb(ci, buf):
      pltpu.async_copy(rows_v.at[buf],
                       out_hbm.at[pl.ds(base0 + ci * _C, _C)], wsems[buf])

    def wait_wb(ci, buf):
      pltpu.make_async_copy(rows_v.at[buf],
                            out_hbm.at[pl.ds(base0 + ci * _C, _C)],
                            wsems[buf]).wait()

    start_gather(0, 0)

    def pair(g, carry):
      a = 2 * g
      wait_gather(a, 0)

      @pl.when(g >= 1)
      def _():
        wait_wb(a - 1, 1)

      start_gather(a + 1, 1)
      start_wb(a, 0)
      wait_gather(a + 1, 1)
      wait_wb(a, 0)

      @pl.when(a + 2 < nch)
      def _():
        start_gather(a + 2, 0)

      start_wb(a + 1, 1)
      return carry

    lax.fori_loop(0, npairs, pair, 0)
    wait_wb(nch - 1, 1)

  return sc_gather


def _ln_body(nseg, eps, s, d, tok_ref, segi_ref, pos_ref, segtab_ref,
             gam_ref, bet_ref, out_ref):
  tok = tok_ref[...].reshape(_BB, s, d)
  pos = pos_ref[...]
  segi = segi_ref[...][:, :, None]
  st = segtab_ref[...]
  h = tok + pos[None, :, :]
  seg_enc = jnp.broadcast_to(st[0][None, None, :], h.shape)
  for k in range(1, nseg):
    seg_enc = jnp.where(segi == k, st[k][None, None, :], seg_enc)
  h = h + seg_enc
  mean = jnp.mean(h, axis=-1, keepdims=True)
  c = h - mean
  var = jnp.mean(c * c, axis=-1, keepdims=True)
  out_ref[...] = (c * lax.rsqrt(var + eps) * gam_ref[...][None, :, :]
                  + bet_ref[...][None, :, :])


def _make_tc_ln(b, s, d, nseg, eps):
  assert b % _BB == 0
  grid = (b // _BB,)
  return pl.pallas_call(
      functools.partial(_ln_body, nseg, eps, s, d),
      grid=grid,
      in_specs=[
          pl.BlockSpec((_BB * s, d), lambda i: (i, 0)),
          pl.BlockSpec((_BB, s), lambda i: (i, 0)),
          pl.BlockSpec((s, d), lambda i: (0, 0)),
          pl.BlockSpec((nseg, d), lambda i: (0, 0)),
          pl.BlockSpec((1, d), lambda i: (0, 0)),
          pl.BlockSpec((1, d), lambda i: (0, 0)),
      ],
      out_specs=pl.BlockSpec((_BB, s, d), lambda i: (i, 0, 0)),
      out_shape=jax.ShapeDtypeStruct((b, s, d), jnp.float32),
      compiler_params=pltpu.CompilerParams(
          dimension_semantics=("arbitrary",)),
  )


def kernel(x, seg, token_table, pos_table, seg_table, gamma, beta):
  b, s = x.shape
  _, d = token_table.shape
  nseg = seg_table.shape[0]
  n = b * s
  xf = x.reshape(_NW, n // (_NW * _C), _C).astype(jnp.int32)
  gathered = _make_sc_gather(n, d)(token_table.astype(jnp.float32), xf)
  segi = seg.astype(jnp.int32)
  out = _make_tc_ln(b, s, d, nseg, 1e-5)(
      gathered, segi, pos_table.astype(jnp.float32),
      seg_table.astype(jnp.float32), gamma.astype(jnp.float32).reshape(1, d),
      beta.astype(jnp.float32).reshape(1, d))
  return out
